# baseline (speedup 1.0000x reference)
import functools

import jax
import jax.numpy as jnp
from jax import lax
from jax.experimental import pallas as pl
from jax.experimental.pallas import tpu as pltpu
from jax.experimental.pallas import tpu_sc as plsc

D0 = 64
D1 = 32
N = 4096
K = 128
NCHUNK = N // K
NV = K // 16

_NUM_TILES = 32
GROUPS_PER_TILE = 8
GROUP_ROWS = 8


def _rev16(v):
  return lax.rev(v, (0,))


def _vsort(v):
  return lax.sort(v, dimension=0, is_stable=False)


def _clean_asc(c):
  m = len(c)
  if m == 1:
    return [_vsort(c[0])]
  h = m // 2
  lo = [jnp.minimum(c[i], c[i + h]) for i in range(h)]
  hi = [jnp.maximum(c[i], c[i + h]) for i in range(h)]
  return _clean_asc(lo) + _clean_asc(hi)


def _build_run(vecs, sign):
  n = len(vecs)
  if n == 1:
    v = vecs[0] if sign > 0 else -vecs[0]
    return [_vsort(v)]
  h = n // 2
  left = _build_run(vecs[:h], sign)
  right = _build_run(vecs[h:], -sign)
  return _clean_asc(left + [-x for x in right])


def _prune_merge(acc, run_neg):
  hi = [jnp.maximum(acc[i], -run_neg[i]) for i in range(NV)]
  return _clean_asc(hi)


def _row_topk2(row_ref, buf, jj):
  def load_chunk(dj, c):
    off = c * K
    return [
        row_ref[buf, jj + dj, pl.ds(off + i * 16, 16)] for i in range(NV)
    ]

  acc_a = _build_run(load_chunk(0, 0), 1)
  acc_b = _build_run(load_chunk(1, 0), 1)

  def chunk_body(c, carry):
    acc_a, acc_b = list(carry[0]), list(carry[1])
    acc_a = _prune_merge(acc_a, _build_run(load_chunk(0, c), -1))
    acc_b = _prune_merge(acc_b, _build_run(load_chunk(1, c), -1))
    return tuple(acc_a), tuple(acc_b)

  acc_a, acc_b = lax.fori_loop(
      1, NCHUNK, chunk_body, (tuple(acc_a), tuple(acc_b)), unroll=False
  )
  return list(acc_a), list(acc_b)


def _group_slice(x_hbm, wid, g):
  i = 2 * wid + g // 4
  j0 = (g % 4) * GROUP_ROWS
  return x_hbm.at[i, pl.ds(j0, GROUP_ROWS)]


def _sc_kernel_body(x_hbm, out_hbm, row_v, out_v, in_sem):
  wid = lax.axis_index("s") * 2 + lax.axis_index("c")

  pltpu.make_async_copy(
      _group_slice(x_hbm, wid, 0), row_v.at[0], in_sem
  ).start()

  def group_body(g, _):
    buf = lax.rem(g, 2)

    @pl.when(g < GROUPS_PER_TILE - 1)
    def _():
      pltpu.make_async_copy(
          _group_slice(x_hbm, wid, g + 1), row_v.at[1 - buf], in_sem
      ).start()

    pltpu.make_async_copy(
        _group_slice(x_hbm, wid, g), row_v.at[buf], in_sem
    ).wait()

    def row_body(p, _):
      jj = 2 * p
      acc_a, acc_b = _row_topk2(row_v, buf, jj)
      for dj, acc in ((0, acc_a), (1, acc_b)):
        r = g * GROUP_ROWS + jj + dj
        i_loc = r // D1
        j = lax.rem(r, D1)
        for l in range(NV):
          out_v[i_loc, j, pl.ds(l * 16, 16)] = _rev16(acc[NV - 1 - l])
      return 0

    lax.fori_loop(0, GROUP_ROWS // 2, row_body, 0, unroll=False)
    return 0

  lax.fori_loop(0, GROUPS_PER_TILE, group_body, 0, unroll=False)

  pltpu.sync_copy(out_v, out_hbm.at[pl.ds(2 * wid, 2)])


_mesh = plsc.VectorSubcoreMesh(core_axis_name="c", subcore_axis_name="s")

_topk_call = functools.partial(
    pl.kernel,
    out_type=jax.ShapeDtypeStruct((D0, D1, K), jnp.float32),
    mesh=_mesh,
    compiler_params=pltpu.CompilerParams(
        needs_layout_passes=False, use_tc_tiling_on_sc=True
    ),
    scratch_types=[
        pltpu.VMEM((2, GROUP_ROWS, N), jnp.float32),
        pltpu.VMEM((2, D1, K), jnp.float32),
        pltpu.SemaphoreType.DMA,
    ],
)(_sc_kernel_body)


@jax.jit
def kernel(x):
  return _topk_call(x)


# speedup vs baseline: 1.0015x; 1.0015x over previous
import functools

import jax
import jax.numpy as jnp
from jax import lax
from jax.experimental import pallas as pl
from jax.experimental.pallas import tpu as pltpu
from jax.experimental.pallas import tpu_sc as plsc

D0 = 64
D1 = 32
N = 4096
K = 128
NCHUNK = N // K
NV = K // 16

_NUM_TILES = 32
GROUPS_PER_TILE = 8
GROUP_ROWS = 8


def _rev16(v):
  return lax.rev(v, (0,))


def _vsort(v):
  return lax.sort(v, dimension=0, is_stable=False)


def _clean_asc(c):
  m = len(c)
  if m == 1:
    return [_vsort(c[0])]
  h = m // 2
  lo = [jnp.minimum(c[i], c[i + h]) for i in range(h)]
  hi = [jnp.maximum(c[i], c[i + h]) for i in range(h)]
  return _clean_asc(lo) + _clean_asc(hi)


def _build_run(vecs, sign):
  n = len(vecs)
  if n == 1:
    v = vecs[0] if sign > 0 else -vecs[0]
    return [_vsort(v)]
  h = n // 2
  left = _build_run(vecs[:h], sign)
  right = _build_run(vecs[h:], -sign)
  return _clean_asc(left + [-x for x in right])


def _prune_merge(acc, run_neg):
  hi = [jnp.maximum(acc[i], -run_neg[i]) for i in range(NV)]
  return _clean_asc(hi)


def _row_topk2(row_ref, buf, jj):
  def load_chunk(dj, c):
    off = c * K
    return [
        row_ref[buf, jj + dj, pl.ds(off + i * 16, 16)] for i in range(NV)
    ]

  acc_a = _build_run(load_chunk(0, 0), 1)
  acc_b = _build_run(load_chunk(1, 0), 1)

  def chunk_body(c, carry):
    acc_a, acc_b = list(carry[0]), list(carry[1])
    acc_a = _prune_merge(acc_a, _build_run(load_chunk(0, c), -1))
    acc_b = _prune_merge(acc_b, _build_run(load_chunk(1, c), -1))
    return tuple(acc_a), tuple(acc_b)

  acc_a, acc_b = lax.fori_loop(
      1, NCHUNK, chunk_body, (tuple(acc_a), tuple(acc_b)), unroll=False
  )
  return list(acc_a), list(acc_b)


def _group_slice(x_hbm, wid, g):
  i = 2 * wid + g // 4
  j0 = (g % 4) * GROUP_ROWS
  return x_hbm.at[i, pl.ds(j0, GROUP_ROWS)]


def _sc_kernel_body(x_hbm, out_hbm, row_v, out_v, in_sem):
  wid = lax.axis_index("s") * 2 + lax.axis_index("c")

  pltpu.make_async_copy(
      _group_slice(x_hbm, wid, 0), row_v.at[0], in_sem
  ).start()

  def group_body(g, _):
    buf = lax.rem(g, 2)

    @pl.when(g < GROUPS_PER_TILE - 1)
    def _():
      pltpu.make_async_copy(
          _group_slice(x_hbm, wid, g + 1), row_v.at[1 - buf], in_sem
      ).start()

    pltpu.make_async_copy(
        _group_slice(x_hbm, wid, g), row_v.at[buf], in_sem
    ).wait()

    def row_body(p):
      jj = 2 * p
      acc_a, acc_b = _row_topk2(row_v, buf, jj)
      for dj, acc in ((0, acc_a), (1, acc_b)):
        r = g * GROUP_ROWS + jj + dj
        i_loc = r // D1
        j = lax.rem(r, D1)
        for l in range(NV):
          out_v[i_loc, j, pl.ds(l * 16, 16)] = _rev16(acc[NV - 1 - l])

    plsc.parallel_loop(0, GROUP_ROWS // 2)(row_body)
    return 0

  lax.fori_loop(0, GROUPS_PER_TILE, group_body, 0, unroll=False)

  pltpu.sync_copy(out_v, out_hbm.at[pl.ds(2 * wid, 2)])


_mesh = plsc.VectorSubcoreMesh(core_axis_name="c", subcore_axis_name="s")

_topk_call = functools.partial(
    pl.kernel,
    out_type=jax.ShapeDtypeStruct((D0, D1, K), jnp.float32),
    mesh=_mesh,
    compiler_params=pltpu.CompilerParams(
        needs_layout_passes=False, use_tc_tiling_on_sc=True
    ),
    scratch_types=[
        pltpu.VMEM((2, GROUP_ROWS, N), jnp.float32),
        pltpu.VMEM((2, D1, K), jnp.float32),
        pltpu.SemaphoreType.DMA,
    ],
)(_sc_kernel_body)


@jax.jit
def kernel(x):
  return _topk_call(x)
